# baseline (speedup 1.0000x reference)
import jax
import jax.numpy as jnp
from jax.experimental import pallas as pl
from jax.experimental.pallas import tpu as pltpu

_B, _V = 8, 1000000
_W = 32768
_NC = 31
_EPS = 1e-10


def _rotl(x, d):
    return (x << jnp.uint32(d)) | (x >> jnp.uint32(32 - d))


def _threefry_bits(idx):
    ks0 = jnp.uint32(0)
    ks1 = jnp.uint32(1)
    ks2 = jnp.uint32(0x1BD11BDA) ^ ks0 ^ ks1
    ks = (ks0, ks1, ks2)
    rots = ((13, 15, 26, 6), (17, 29, 16, 24))
    x0 = jnp.zeros_like(idx) + ks0
    x1 = idx + ks1
    for i in range(5):
        for r in rots[i % 2]:
            x0 = x0 + x1
            x1 = _rotl(x1, r) ^ x0
        x0 = x0 + ks[(i + 1) % 3]
        x1 = x1 + ks[(i + 2) % 3] + jnp.uint32(i + 1)
    return x0 ^ x1


def _kernel(a_ref, y_ref, y2_ref, ss_ref,
            g_s, mg_s, ma_s, sg_s, sa_s):
    p = pl.program_id(0)
    k = pl.program_id(1)
    eps = jnp.float32(_EPS)
    ninf = jnp.float32(-jnp.inf)

    cols = pl.ds(k * _W, _W)

    @pl.when(p == 0)
    def _p0():
        @pl.when(k == 0)
        def _init():
            mg_s[...] = jnp.full((8, 128), ninf, jnp.float32)
            ma_s[...] = jnp.full((8, 128), ninf, jnp.float32)
            sg_s[...] = jnp.zeros((8, 128), jnp.float32)
            sa_s[...] = jnp.zeros((8, 128), jnp.float32)

        lane = jax.lax.broadcasted_iota(jnp.int32, (8, _W), 1)
        mask = (k * _W + lane) < _V
        a = jnp.where(mask, a_ref[...], ninf)
        ridx = jax.lax.broadcasted_iota(jnp.uint32, (8, _W), 0) * jnp.uint32(_V)
        idx = ridx + lane.astype(jnp.uint32) + jnp.uint32(_W) * k.astype(jnp.uint32)
        bits = _threefry_bits(idx)
        u = jax.lax.bitcast_convert_type(
            (bits >> jnp.uint32(9)) | jnp.uint32(0x3F800000), jnp.float32
        ) - jnp.float32(1.0)
        g = a - jnp.log(eps - jnp.log(u + eps))
        g_s[:, cols] = g
        mg_s[...] = jnp.maximum(
            mg_s[...], jnp.max(g, axis=1, keepdims=True))
        ma_s[...] = jnp.maximum(
            ma_s[...], jnp.max(a, axis=1, keepdims=True))

    @pl.when(p == 1)
    def _p1():
        mg = mg_s[...][:, :1]
        ma = ma_s[...][:, :1]
        e = jnp.exp(g_s[:, cols] - mg)
        g_s[:, cols] = e
        sg_s[...] = sg_s[...] + jnp.sum(e, axis=1, keepdims=True)
        lane = jax.lax.broadcasted_iota(jnp.int32, (8, _W), 1)
        mask = (k * _W + lane) < _V
        a = jnp.where(mask, a_ref[...], ninf)
        e2 = jnp.exp(a - ma)
        sa_s[...] = sa_s[...] + jnp.sum(e2, axis=1, keepdims=True)

    @pl.when(p == 2)
    def _p2():
        rg = jnp.float32(1.0) / sg_s[...][:, :1]
        ra = jnp.float32(1.0) / sa_s[...][:, :1]
        y = jnp.maximum(g_s[:, cols] * rg, eps)
        y_ref[...] = y
        y2_ref[...] = y
        ma = ma_s[...][:, :1]
        ss_ref[...] = jnp.exp(a_ref[...] - ma) * ra


def kernel(alpha):
    y, y2, ss = pl.pallas_call(
        _kernel,
        grid=(3, _NC),
        in_specs=[pl.BlockSpec((8, _W), lambda p, k: (0, k))],
        out_specs=[
            pl.BlockSpec((8, _W), lambda p, k: (0, (p == 2) * k)),
            pl.BlockSpec((8, _W), lambda p, k: (0, (p == 2) * k)),
            pl.BlockSpec((8, _W), lambda p, k: (0, (p == 2) * k)),
        ],
        out_shape=[
            jax.ShapeDtypeStruct((_B, _V), jnp.float32),
            jax.ShapeDtypeStruct((_B, _V), jnp.float32),
            jax.ShapeDtypeStruct((_B, _V), jnp.float32),
        ],
        scratch_shapes=[
            pltpu.VMEM((8, _NC * _W), jnp.float32),
            pltpu.VMEM((8, 128), jnp.float32),
            pltpu.VMEM((8, 128), jnp.float32),
            pltpu.VMEM((8, 128), jnp.float32),
            pltpu.VMEM((8, 128), jnp.float32),
        ],
    )(alpha)
    return (y, y2, ss)


# speedup vs baseline: 1.4795x; 1.4795x over previous
import jax
import jax.numpy as jnp
from jax.experimental import pallas as pl
from jax.experimental.pallas import tpu as pltpu

_B, _V = 8, 1000000
_W = 32768
_NC = 31
_EPS = 1e-10


def _rotl(x, d):
    return (x << jnp.uint32(d)) | (x >> jnp.uint32(32 - d))


def _threefry_bits(idx):
    ks0 = jnp.uint32(0)
    ks1 = jnp.uint32(1)
    ks2 = jnp.uint32(0x1BD11BDA) ^ ks0 ^ ks1
    ks = (ks0, ks1, ks2)
    rots = ((13, 15, 26, 6), (17, 29, 16, 24))
    x0 = jnp.zeros_like(idx) + ks0
    x1 = idx + ks1
    for i in range(5):
        for r in rots[i % 2]:
            x0 = x0 + x1
            x1 = _rotl(x1, r) ^ x0
        x0 = x0 + ks[(i + 1) % 3]
        x1 = x1 + ks[(i + 2) % 3] + jnp.uint32(i + 1)
    return x0 ^ x1


def _kernel(a_ref, y_ref, y2_ref, ss_ref,
            g_s, mga, maa, sga, saa, mg_s, ma_s, rg_s, ra_s, base_s):
    p = pl.program_id(0)
    k = pl.program_id(1)
    eps = jnp.float32(_EPS)
    ninf = jnp.float32(-jnp.inf)

    cols = pl.ds(k * _W, _W)

    def _masked_a():
        lane = jax.lax.broadcasted_iota(jnp.int32, (8, _W), 1)
        mask = (k * _W + lane) < _V
        return jnp.where(mask, a_ref[...], ninf)

    def _p0_body(masked):
        a = _masked_a() if masked else a_ref[...]
        idx = base_s[...] + jnp.uint32(_W) * k.astype(jnp.uint32)
        bits = _threefry_bits(idx)
        u = jax.lax.bitcast_convert_type(
            (bits >> jnp.uint32(9)) | jnp.uint32(0x3F800000), jnp.float32
        ) - jnp.float32(1.0)
        g = a - jnp.log(eps - jnp.log(u + eps))
        g_s[:, cols] = g
        mga[...] = jnp.maximum(mga[...], g)
        maa[...] = jnp.maximum(maa[...], a)

    @pl.when((p == 0) & (k == 0))
    def _init():
        mga[...] = jnp.full((8, _W), ninf, jnp.float32)
        maa[...] = jnp.full((8, _W), ninf, jnp.float32)
        base_s[...] = (
            jax.lax.broadcasted_iota(jnp.uint32, (8, _W), 0) * jnp.uint32(_V)
            + jax.lax.broadcasted_iota(jnp.uint32, (8, _W), 1))

    @pl.when((p == 0) & (k < _NC - 1))
    def _p0():
        _p0_body(False)

    @pl.when((p == 0) & (k == _NC - 1))
    def _p0t():
        _p0_body(True)

    def _p1_body(masked):
        mg = mg_s[...][:, :1]
        ma = ma_s[...][:, :1]
        e = jnp.exp(g_s[:, cols] - mg)
        g_s[:, cols] = e
        sga[...] = sga[...] + e
        a = _masked_a() if masked else a_ref[...]
        e2 = jnp.exp(a - ma)
        saa[...] = saa[...] + e2

    @pl.when((p == 1) & (k == 0))
    def _stats():
        mg_s[...] = jnp.broadcast_to(
            jnp.max(mga[...], axis=1, keepdims=True), (8, 128))
        ma_s[...] = jnp.broadcast_to(
            jnp.max(maa[...], axis=1, keepdims=True), (8, 128))
        sga[...] = jnp.zeros((8, _W), jnp.float32)
        saa[...] = jnp.zeros((8, _W), jnp.float32)

    @pl.when((p == 1) & (k < _NC - 1))
    def _p1():
        _p1_body(False)

    @pl.when((p == 1) & (k == _NC - 1))
    def _p1t():
        _p1_body(True)

    @pl.when(p == 2)
    def _p2():
        @pl.when(k == 0)
        def _recip():
            rg_s[...] = jnp.broadcast_to(
                jnp.float32(1.0)
                / jnp.sum(sga[...], axis=1, keepdims=True), (8, 128))
            ra_s[...] = jnp.broadcast_to(
                jnp.float32(1.0)
                / jnp.sum(saa[...], axis=1, keepdims=True), (8, 128))

        rg = rg_s[...][:, :1]
        ra = ra_s[...][:, :1]
        y = jnp.maximum(g_s[:, cols] * rg, eps)
        y_ref[...] = y
        y2_ref[...] = y
        ma = ma_s[...][:, :1]
        ss_ref[...] = jnp.exp(a_ref[...] - ma) * ra


def kernel(alpha):
    y, y2, ss = pl.pallas_call(
        _kernel,
        grid=(3, _NC),
        in_specs=[pl.BlockSpec((8, _W), lambda p, k: (0, k))],
        out_specs=[
            pl.BlockSpec((8, _W), lambda p, k: (0, (p == 2) * k)),
            pl.BlockSpec((8, _W), lambda p, k: (0, (p == 2) * k)),
            pl.BlockSpec((8, _W), lambda p, k: (0, (p == 2) * k)),
        ],
        out_shape=[
            jax.ShapeDtypeStruct((_B, _V), jnp.float32),
            jax.ShapeDtypeStruct((_B, _V), jnp.float32),
            jax.ShapeDtypeStruct((_B, _V), jnp.float32),
        ],
        scratch_shapes=[
            pltpu.VMEM((8, _NC * _W), jnp.float32),
            pltpu.VMEM((8, _W), jnp.float32),
            pltpu.VMEM((8, _W), jnp.float32),
            pltpu.VMEM((8, _W), jnp.float32),
            pltpu.VMEM((8, _W), jnp.float32),
            pltpu.VMEM((8, 128), jnp.float32),
            pltpu.VMEM((8, 128), jnp.float32),
            pltpu.VMEM((8, 128), jnp.float32),
            pltpu.VMEM((8, 128), jnp.float32),
            pltpu.VMEM((8, _W), jnp.uint32),
        ],
    )(alpha)
    return (y, y2, ss)


# speedup vs baseline: 1.4970x; 1.0119x over previous
import jax
import jax.numpy as jnp
from jax.experimental import pallas as pl
from jax.experimental.pallas import tpu as pltpu

_B, _V = 8, 1000000
_W = 32768
_NC = 31
_EPS = 1e-10


def _rotl(x, d):
    return (x << jnp.uint32(d)) | (x >> jnp.uint32(32 - d))


def _threefry_bits(idx):
    ks0 = jnp.uint32(0)
    ks1 = jnp.uint32(1)
    ks2 = jnp.uint32(0x1BD11BDA) ^ ks0 ^ ks1
    ks = (ks0, ks1, ks2)
    rots = ((13, 15, 26, 6), (17, 29, 16, 24))
    x0 = jnp.zeros_like(idx) + ks0
    x1 = idx + ks1
    for i in range(5):
        for r in rots[i % 2]:
            x0 = x0 + x1
            x1 = _rotl(x1, r) ^ x0
        x0 = x0 + ks[(i + 1) % 3]
        x1 = x1 + ks[(i + 2) % 3] + jnp.uint32(i + 1)
    return x0 ^ x1


_AW = 1024


def _fold(x, op):
    w = x.shape[1]
    while w > _AW:
        w //= 2
        x = op(x[:, :w], x[:, w:2 * w])
    return x


def _kernel(a_ref, y_ref, y2_ref, ss_ref,
            g_s, mga, maa, sga, saa, mg_s, ma_s, rg_s, ra_s, base_s):
    p = pl.program_id(0)
    k = pl.program_id(1)
    eps = jnp.float32(_EPS)
    ninf = jnp.float32(-jnp.inf)

    cols = pl.ds(k * _W, _W)

    def _masked_a():
        lane = jax.lax.broadcasted_iota(jnp.int32, (8, _W), 1)
        mask = (k * _W + lane) < _V
        return jnp.where(mask, a_ref[...], ninf)

    def _p0_body(masked):
        a = _masked_a() if masked else a_ref[...]
        idx = base_s[...] + jnp.uint32(_W) * k.astype(jnp.uint32)
        bits = _threefry_bits(idx)
        u = jax.lax.bitcast_convert_type(
            (bits >> jnp.uint32(9)) | jnp.uint32(0x3F800000), jnp.float32
        ) - jnp.float32(1.0)
        g = a - jnp.log(eps - jnp.log(u + eps))
        g_s[:, cols] = g
        mga[...] = jnp.maximum(mga[...], _fold(g, jnp.maximum))
        maa[...] = jnp.maximum(maa[...], _fold(a, jnp.maximum))

    @pl.when((p == 0) & (k == 0))
    def _init():
        mga[...] = jnp.full((8, _AW), ninf, jnp.float32)
        maa[...] = jnp.full((8, _AW), ninf, jnp.float32)
        base_s[...] = (
            jax.lax.broadcasted_iota(jnp.uint32, (8, _W), 0) * jnp.uint32(_V)
            + jax.lax.broadcasted_iota(jnp.uint32, (8, _W), 1))

    @pl.when((p == 0) & (k < _NC - 1))
    def _p0():
        _p0_body(False)

    @pl.when((p == 0) & (k == _NC - 1))
    def _p0t():
        _p0_body(True)

    def _p1_body(masked):
        mg = mg_s[...][:, :1]
        ma = ma_s[...][:, :1]
        e = jnp.exp(g_s[:, cols] - mg)
        g_s[:, cols] = e
        sga[...] = sga[...] + _fold(e, jnp.add)
        a = _masked_a() if masked else a_ref[...]
        e2 = jnp.exp(a - ma)
        saa[...] = saa[...] + _fold(e2, jnp.add)

    @pl.when((p == 1) & (k == 0))
    def _stats():
        mg_s[...] = jnp.broadcast_to(
            jnp.max(mga[...], axis=1, keepdims=True), (8, 128))
        ma_s[...] = jnp.broadcast_to(
            jnp.max(maa[...], axis=1, keepdims=True), (8, 128))
        sga[...] = jnp.zeros((8, _AW), jnp.float32)
        saa[...] = jnp.zeros((8, _AW), jnp.float32)

    @pl.when((p == 1) & (k < _NC - 1))
    def _p1():
        _p1_body(False)

    @pl.when((p == 1) & (k == _NC - 1))
    def _p1t():
        _p1_body(True)

    @pl.when(p == 2)
    def _p2():
        @pl.when(k == 0)
        def _recip():
            rg_s[...] = jnp.broadcast_to(
                jnp.float32(1.0)
                / jnp.sum(sga[...], axis=1, keepdims=True), (8, 128))
            ra_s[...] = jnp.broadcast_to(
                jnp.float32(1.0)
                / jnp.sum(saa[...], axis=1, keepdims=True), (8, 128))

        rg = rg_s[...][:, :1]
        ra = ra_s[...][:, :1]
        y = jnp.maximum(g_s[:, cols] * rg, eps)
        y_ref[...] = y
        y2_ref[...] = y
        ma = ma_s[...][:, :1]
        ss_ref[...] = jnp.exp(a_ref[...] - ma) * ra


def kernel(alpha):
    y, y2, ss = pl.pallas_call(
        _kernel,
        grid=(3, _NC),
        in_specs=[pl.BlockSpec((8, _W), lambda p, k: (0, k))],
        out_specs=[
            pl.BlockSpec((8, _W), lambda p, k: (0, (p == 2) * k)),
            pl.BlockSpec((8, _W), lambda p, k: (0, (p == 2) * k)),
            pl.BlockSpec((8, _W), lambda p, k: (0, (p == 2) * k)),
        ],
        out_shape=[
            jax.ShapeDtypeStruct((_B, _V), jnp.float32),
            jax.ShapeDtypeStruct((_B, _V), jnp.float32),
            jax.ShapeDtypeStruct((_B, _V), jnp.float32),
        ],
        scratch_shapes=[
            pltpu.VMEM((8, _NC * _W), jnp.float32),
            pltpu.VMEM((8, _AW), jnp.float32),
            pltpu.VMEM((8, _AW), jnp.float32),
            pltpu.VMEM((8, _AW), jnp.float32),
            pltpu.VMEM((8, _AW), jnp.float32),
            pltpu.VMEM((8, 128), jnp.float32),
            pltpu.VMEM((8, 128), jnp.float32),
            pltpu.VMEM((8, 128), jnp.float32),
            pltpu.VMEM((8, 128), jnp.float32),
            pltpu.VMEM((8, _W), jnp.uint32),
        ],
    )(alpha)
    return (y, y2, ss)
